# baseline (speedup 1.0000x reference)
import numpy as np
import jax
import jax.numpy as jnp
from jax.experimental import pallas as pl
from jax.experimental.pallas import tpu as pltpu

_H = 12
_DH = 64
_K = 256
_BQ = 256

_INT_MIN = np.int32(-(2 ** 31))
_INT_LOW31 = np.int32(0x7FFFFFFF)


def _proj_kernel(x_ref, w_ref, o_ref):
    o_ref[...] = jnp.dot(x_ref[...], w_ref[...],
                         preferred_element_type=jnp.float32)


def _attn_kernel(q_ref, k_ref, v_ref, o_ref):
    q = q_ref[0]
    k = k_ref[0]
    s = jnp.dot(q, k.T, preferred_element_type=jnp.float32) * (1.0 / 8.0)

    bq = q.shape[0]
    t = jnp.zeros((bq, 1), jnp.int32)
    for b in range(31, -1, -1):
        m = (1 << b) if b < 31 else ((1 << 31) - (1 << 32))
        t_try = t | np.int32(m)
        skt = t_try ^ _INT_MIN
        fbits = jnp.where(skt < 0, skt ^ _INT_LOW31, skt)
        fthr = jax.lax.bitcast_convert_type(fbits, jnp.float32)
        cnt = jnp.sum((s >= fthr).astype(jnp.int32), axis=1, keepdims=True)
        t = jnp.where(cnt >= _K, t_try, t)
    skt = t ^ _INT_MIN
    fbits = jnp.where(skt < 0, skt ^ _INT_LOW31, skt)
    fthr = jax.lax.bitcast_convert_type(fbits, jnp.float32)

    sm = jnp.where(s >= fthr, s, jnp.float32(-1e30))
    mx = jnp.max(sm, axis=1, keepdims=True)
    p = jnp.exp(sm - mx)
    l = jnp.sum(p, axis=1, keepdims=True)
    o = jnp.dot(p, v_ref[0], preferred_element_type=jnp.float32) / l
    o_ref[0] = o


def kernel(x, Wq, Wk, Wv, Wo):
    B, S, D = x.shape
    x2 = x.reshape(S, D)
    Wqkv = jnp.concatenate([Wq, Wk, Wv], axis=1)

    qkv = pl.pallas_call(
        _proj_kernel,
        grid=(S // _BQ,),
        in_specs=[pl.BlockSpec((_BQ, D), lambda i: (i, 0)),
                  pl.BlockSpec((D, 3 * D), lambda i: (0, 0))],
        out_specs=pl.BlockSpec((_BQ, 3 * D), lambda i: (i, 0)),
        out_shape=jax.ShapeDtypeStruct((S, 3 * D), jnp.float32),
    )(x2, Wqkv)

    q = qkv[:, :D].reshape(S, _H, _DH).transpose(1, 0, 2)
    k = qkv[:, D:2 * D].reshape(S, _H, _DH).transpose(1, 0, 2)
    v = qkv[:, 2 * D:].reshape(S, _H, _DH).transpose(1, 0, 2)

    attn = pl.pallas_call(
        _attn_kernel,
        grid=(_H, S // _BQ),
        in_specs=[pl.BlockSpec((1, _BQ, _DH), lambda h, i: (h, i, 0)),
                  pl.BlockSpec((1, S, _DH), lambda h, i: (h, 0, 0)),
                  pl.BlockSpec((1, S, _DH), lambda h, i: (h, 0, 0))],
        out_specs=pl.BlockSpec((1, _BQ, _DH), lambda h, i: (h, i, 0)),
        out_shape=jax.ShapeDtypeStruct((_H, S, _DH), jnp.float32),
        compiler_params=pltpu.CompilerParams(
            dimension_semantics=("parallel", "parallel")),
    )(q, k, v)
    attn = attn.transpose(1, 0, 2).reshape(S, D)

    out = pl.pallas_call(
        _proj_kernel,
        grid=(S // _BQ,),
        in_specs=[pl.BlockSpec((_BQ, D), lambda i: (i, 0)),
                  pl.BlockSpec((D, D), lambda i: (0, 0))],
        out_specs=pl.BlockSpec((_BQ, D), lambda i: (i, 0)),
        out_shape=jax.ShapeDtypeStruct((S, D), jnp.float32),
    )(attn, Wo)

    return out.reshape(B, S, D)


# speedup vs baseline: 1.1507x; 1.1507x over previous
import numpy as np
import jax
import jax.numpy as jnp
from jax.experimental import pallas as pl
from jax.experimental.pallas import tpu as pltpu

_H = 12
_DH = 64
_K = 256
_BQ = 256

_INT_MIN = np.int32(-(2 ** 31))
_INT_LOW31 = np.int32(0x7FFFFFFF)


def _proj_kernel(x_ref, w_ref, o_ref):
    o_ref[...] = jnp.dot(x_ref[...], w_ref[...],
                         preferred_element_type=jnp.float32)


def _attn_kernel(q_ref, k_ref, v_ref, o_ref):
    q = q_ref[0]
    k = k_ref[0]
    s = jnp.dot(q, k.T, preferred_element_type=jnp.float32) * (1.0 / 8.0)

    bits = jax.lax.bitcast_convert_type(s, jnp.int32)
    key = jnp.where(bits < 0, bits ^ _INT_LOW31, bits)

    bq = q.shape[0]
    t = jnp.zeros((bq, 1), jnp.int32)
    for b in range(31, -1, -1):
        m = (1 << b) if b < 31 else ((1 << 31) - (1 << 32))
        t_try = t | np.int32(m)
        st = t_try ^ _INT_MIN
        cnt = jnp.sum((key >= st).astype(jnp.float32), axis=1, keepdims=True)
        t = jnp.where(cnt >= jnp.float32(_K), t_try, t)
    skt = t ^ _INT_MIN

    sm = jnp.where(key >= skt, s, jnp.float32(-1e30))
    mx = jnp.max(sm, axis=1, keepdims=True)
    p = jnp.exp(sm - mx)
    l = jnp.sum(p, axis=1, keepdims=True)
    o = jnp.dot(p, v_ref[0], preferred_element_type=jnp.float32) / l
    o_ref[0] = o


def kernel(x, Wq, Wk, Wv, Wo):
    B, S, D = x.shape
    x2 = x.reshape(S, D)
    Wqkv = jnp.concatenate([Wq, Wk, Wv], axis=1)

    qkv = pl.pallas_call(
        _proj_kernel,
        grid=(S // _BQ,),
        in_specs=[pl.BlockSpec((_BQ, D), lambda i: (i, 0)),
                  pl.BlockSpec((D, 3 * D), lambda i: (0, 0))],
        out_specs=pl.BlockSpec((_BQ, 3 * D), lambda i: (i, 0)),
        out_shape=jax.ShapeDtypeStruct((S, 3 * D), jnp.float32),
    )(x2, Wqkv)

    q = qkv[:, :D].reshape(S, _H, _DH).transpose(1, 0, 2)
    k = qkv[:, D:2 * D].reshape(S, _H, _DH).transpose(1, 0, 2)
    v = qkv[:, 2 * D:].reshape(S, _H, _DH).transpose(1, 0, 2)

    attn = pl.pallas_call(
        _attn_kernel,
        grid=(_H, S // _BQ),
        in_specs=[pl.BlockSpec((1, _BQ, _DH), lambda h, i: (h, i, 0)),
                  pl.BlockSpec((1, S, _DH), lambda h, i: (h, 0, 0)),
                  pl.BlockSpec((1, S, _DH), lambda h, i: (h, 0, 0))],
        out_specs=pl.BlockSpec((1, _BQ, _DH), lambda h, i: (h, i, 0)),
        out_shape=jax.ShapeDtypeStruct((_H, S, _DH), jnp.float32),
        compiler_params=pltpu.CompilerParams(
            dimension_semantics=("parallel", "parallel")),
    )(q, k, v)
    attn = attn.transpose(1, 0, 2).reshape(S, D)

    out = pl.pallas_call(
        _proj_kernel,
        grid=(S // _BQ,),
        in_specs=[pl.BlockSpec((_BQ, D), lambda i: (i, 0)),
                  pl.BlockSpec((D, D), lambda i: (0, 0))],
        out_specs=pl.BlockSpec((_BQ, D), lambda i: (i, 0)),
        out_shape=jax.ShapeDtypeStruct((S, D), jnp.float32),
    )(attn, Wo)

    return out.reshape(B, S, D)


# speedup vs baseline: 1.2914x; 1.1223x over previous
import numpy as np
import jax
import jax.numpy as jnp
from jax.experimental import pallas as pl
from jax.experimental.pallas import tpu as pltpu

_H = 12
_DH = 64
_K = 256
_BQ = 256

_INT_MIN = np.int32(-(2 ** 31))
_INT_LOW31 = np.int32(0x7FFFFFFF)


def _proj_kernel(x_ref, w_ref, o_ref):
    o_ref[...] = jnp.dot(x_ref[...], w_ref[...],
                         preferred_element_type=jnp.float32)


def _select_threshold(key):
    bq = key.shape[0]
    t = jnp.zeros((bq, 1), jnp.int32)
    for b in range(31, -1, -1):
        m = (1 << b) if b < 31 else ((1 << 31) - (1 << 32))
        t_try = t | np.int32(m)
        st = t_try ^ _INT_MIN
        cnt = jnp.sum((key >= st).astype(jnp.float32), axis=1, keepdims=True)
        t = jnp.where(cnt >= jnp.float32(_K), t_try, t)
    return t ^ _INT_MIN


def _attn_kernel(q_ref, k_ref, v_ref, o_ref):
    q2 = q_ref[...]
    k2 = k_ref[...]
    v2 = v_ref[...]

    s0 = jax.lax.dot_general(q2[:, :_DH], k2[:, :_DH],
                             (((1,), (1,)), ((), ())),
                             preferred_element_type=jnp.float32)
    s1 = jax.lax.dot_general(q2[:, _DH:], k2[:, _DH:],
                             (((1,), (1,)), ((), ())),
                             preferred_element_type=jnp.float32)
    s = jnp.concatenate([s0, s1], axis=0) * (1.0 / 8.0)

    bits = jax.lax.bitcast_convert_type(s, jnp.int32)
    key = jnp.where(bits < 0, bits ^ _INT_LOW31, bits)

    skt = _select_threshold(key)

    sm = jnp.where(key >= skt, s, jnp.float32(-1e30))
    mx = jnp.max(sm, axis=1, keepdims=True)
    p = jnp.exp(sm - mx)
    l = jnp.sum(p, axis=1, keepdims=True)
    p = p / l

    bq = q2.shape[0]
    o0 = jnp.dot(p[:bq], v2[:, :_DH], preferred_element_type=jnp.float32)
    o1 = jnp.dot(p[bq:], v2[:, _DH:], preferred_element_type=jnp.float32)
    o_ref[...] = jnp.concatenate([o0, o1], axis=1)


def kernel(x, Wq, Wk, Wv, Wo):
    B, S, D = x.shape
    x2 = x.reshape(S, D)
    Wqkv = jnp.concatenate([Wq, Wk, Wv], axis=1)

    qkv = pl.pallas_call(
        _proj_kernel,
        grid=(S // _BQ,),
        in_specs=[pl.BlockSpec((_BQ, D), lambda i: (i, 0)),
                  pl.BlockSpec((D, 3 * D), lambda i: (0, 0))],
        out_specs=pl.BlockSpec((_BQ, 3 * D), lambda i: (i, 0)),
        out_shape=jax.ShapeDtypeStruct((S, 3 * D), jnp.float32),
    )(x2, Wqkv)

    npairs = _H // 2
    hpd = D // 128

    attn = pl.pallas_call(
        _attn_kernel,
        grid=(npairs, S // _BQ),
        in_specs=[pl.BlockSpec((_BQ, 128), lambda g, i: (i, g)),
                  pl.BlockSpec((S, 128), lambda g, i: (0, hpd + g)),
                  pl.BlockSpec((S, 128), lambda g, i: (0, 2 * hpd + g))],
        out_specs=pl.BlockSpec((_BQ, 128), lambda g, i: (i, g)),
        out_shape=jax.ShapeDtypeStruct((S, D), jnp.float32),
        compiler_params=pltpu.CompilerParams(
            dimension_semantics=("parallel", "parallel")),
    )(qkv, qkv, qkv)

    out = pl.pallas_call(
        _proj_kernel,
        grid=(S // _BQ,),
        in_specs=[pl.BlockSpec((_BQ, D), lambda i: (i, 0)),
                  pl.BlockSpec((D, D), lambda i: (0, 0))],
        out_specs=pl.BlockSpec((_BQ, D), lambda i: (i, 0)),
        out_shape=jax.ShapeDtypeStruct((S, D), jnp.float32),
    )(attn, Wo)

    return out.reshape(B, S, D)


# speedup vs baseline: 1.3119x; 1.0159x over previous
import numpy as np
import jax
import jax.numpy as jnp
from jax.experimental import pallas as pl
from jax.experimental.pallas import tpu as pltpu

_H = 12
_DH = 64
_K = 256
_BQ = 256

_INT_MIN = np.int32(-(2 ** 31))
_INT_LOW31 = np.int32(0x7FFFFFFF)


def _proj_kernel(x_ref, w_ref, o_ref):
    o_ref[...] = jnp.dot(x_ref[...], w_ref[...],
                         preferred_element_type=jnp.float32)


def _qkv_kernel(x_ref, wq_ref, wk_ref, wv_ref, q_ref, k_ref, v_ref):
    x = x_ref[...]
    q_ref[...] = jnp.dot(x, wq_ref[...], preferred_element_type=jnp.float32)
    k_ref[...] = jnp.dot(x, wk_ref[...], preferred_element_type=jnp.float32)
    v_ref[...] = jnp.dot(x, wv_ref[...], preferred_element_type=jnp.float32)


def _select_threshold(key):
    bq = key.shape[0]
    t = jnp.zeros((bq, 1), jnp.int32)
    for b in range(31, -1, -1):
        m = (1 << b) if b < 31 else ((1 << 31) - (1 << 32))
        t_try = t | np.int32(m)
        st = t_try ^ _INT_MIN
        cnt = jnp.sum((key >= st).astype(jnp.float32), axis=1, keepdims=True)
        t = jnp.where(cnt >= jnp.float32(_K), t_try, t)
    return t ^ _INT_MIN


def _attn_kernel(q_ref, k_ref, v_ref, o_ref):
    q2 = q_ref[...]
    k2 = k_ref[...]
    v2 = v_ref[...]

    s0 = jax.lax.dot_general(q2[:, :_DH], k2[:, :_DH],
                             (((1,), (1,)), ((), ())),
                             preferred_element_type=jnp.float32)
    s1 = jax.lax.dot_general(q2[:, _DH:], k2[:, _DH:],
                             (((1,), (1,)), ((), ())),
                             preferred_element_type=jnp.float32)
    s = jnp.concatenate([s0, s1], axis=0) * (1.0 / 8.0)

    bits = jax.lax.bitcast_convert_type(s, jnp.int32)
    key = jnp.where(bits < 0, bits ^ _INT_LOW31, bits)

    skt = _select_threshold(key)

    sm = jnp.where(key >= skt, s, jnp.float32(-1e30))
    mx = jnp.max(sm, axis=1, keepdims=True)
    p = jnp.exp(sm - mx)
    l = jnp.sum(p, axis=1, keepdims=True)

    bq = q2.shape[0]
    o0 = jnp.dot(p[:bq], v2[:, :_DH], preferred_element_type=jnp.float32)
    o1 = jnp.dot(p[bq:], v2[:, _DH:], preferred_element_type=jnp.float32)
    o_ref[...] = jnp.concatenate([o0 / l[:bq], o1 / l[bq:]], axis=1)


def kernel(x, Wq, Wk, Wv, Wo):
    B, S, D = x.shape
    x2 = x.reshape(S, D)

    wspec = pl.BlockSpec((D, D), lambda i: (0, 0))
    ospec = pl.BlockSpec((_BQ, D), lambda i: (i, 0))
    oshape = jax.ShapeDtypeStruct((S, D), jnp.float32)
    q, k, v = pl.pallas_call(
        _qkv_kernel,
        grid=(S // _BQ,),
        in_specs=[pl.BlockSpec((_BQ, D), lambda i: (i, 0)),
                  wspec, wspec, wspec],
        out_specs=[ospec, ospec, ospec],
        out_shape=[oshape, oshape, oshape],
    )(x2, Wq, Wk, Wv)

    npairs = _H // 2

    attn = pl.pallas_call(
        _attn_kernel,
        grid=(npairs, S // _BQ),
        in_specs=[pl.BlockSpec((_BQ, 128), lambda g, i: (i, g)),
                  pl.BlockSpec((S, 128), lambda g, i: (0, g)),
                  pl.BlockSpec((S, 128), lambda g, i: (0, g))],
        out_specs=pl.BlockSpec((_BQ, 128), lambda g, i: (i, g)),
        out_shape=jax.ShapeDtypeStruct((S, D), jnp.float32),
        compiler_params=pltpu.CompilerParams(
            dimension_semantics=("parallel", "parallel")),
    )(q, k, v)

    out = pl.pallas_call(
        _proj_kernel,
        grid=(S // _BQ,),
        in_specs=[pl.BlockSpec((_BQ, D), lambda i: (i, 0)),
                  pl.BlockSpec((D, D), lambda i: (0, 0))],
        out_specs=pl.BlockSpec((_BQ, D), lambda i: (i, 0)),
        out_shape=jax.ShapeDtypeStruct((S, D), jnp.float32),
    )(attn, Wo)

    return out.reshape(B, S, D)
